# Initial kernel scaffold; baseline (speedup 1.0000x reference)
import jax
import jax.numpy as jnp
from jax.experimental import pallas as pl


def kernel(x, edge_index, edge_weight, W1, b1, W2, b2, W3, b3, Wp1, bp1, Wp2, bp2):
    raise NotImplementedError("write your pallas kernel here")


# speedup vs baseline: 3.5934x; 3.5934x over previous
import functools

import jax
import jax.numpy as jnp
from jax import lax
from jax.experimental import pallas as pl
from jax.experimental.pallas import tpu as pltpu
from jax.experimental.pallas import tpu_sc as plsc

N = 10000
NP = 10240
E = 320000
D = 128
NC = 2
NS = 16
NW = NC * NS
EPT = E // NW
CH = 80
NCHUNK = EPT // CH
ROWS_PER_TILE = NP // NS


def _sc_agg_body(h_hbm, row3, col3, w3, zeros_hbm, out_hbm,
                 idxr, idxc, wch, rows, agg, sem):
    cid = lax.axis_index("c")
    sid = lax.axis_index("s")
    wid = cid * NS + sid

    stripe = pl.ds(sid * ROWS_PER_TILE, ROWS_PER_TILE)
    pltpu.sync_copy(zeros_hbm.at[stripe], agg.at[stripe])
    plsc.subcore_barrier()

    def chunk_body(k, carry):
        pltpu.sync_copy(row3.at[wid, k], idxr)
        pltpu.sync_copy(col3.at[wid, k], idxc)
        pltpu.sync_copy(w3.at[wid, k], wch)
        pltpu.async_copy(h_hbm.at[idxr], rows, sem).wait()

        def group_body(g, c2):
            wv = wch[pl.ds(g * 16, 16)]
            for e16 in range(16):
                e = g * 16 + e16
                w = jnp.full((16,), 0.0, jnp.float32) + wv[e16]
                for j in range(D // 16):
                    sl = pl.ds(j * 16, 16)
                    rows[e, sl] = rows[e, sl] * w
            return c2

        lax.fori_loop(0, CH // 16, group_body, 0)
        pltpu.sync_copy(rows, agg.at[idxc], add=True)
        return carry

    lax.fori_loop(0, NCHUNK, chunk_body, 0)
    plsc.subcore_barrier()
    pltpu.sync_copy(agg.at[stripe], out_hbm.at[cid, stripe])


def _sc_aggregate(h, row3, col3, w3, zeros):
    mesh = plsc.VectorSubcoreMesh(core_axis_name="c", subcore_axis_name="s")
    f = pl.kernel(
        _sc_agg_body,
        out_type=jax.ShapeDtypeStruct((NC, NP, D), jnp.float32),
        mesh=mesh,
        scratch_types=[
            pltpu.VMEM((CH,), jnp.int32),
            pltpu.VMEM((CH,), jnp.int32),
            pltpu.VMEM((CH,), jnp.float32),
            pltpu.VMEM((CH, D), jnp.float32),
            pltpu.VMEM_SHARED((NP, D), jnp.float32),
            pltpu.SemaphoreType.DMA,
        ],
    )
    return f(h, row3, col3, w3, zeros)


def _tc_conv_body(p_ref, wt_ref, b_ref, o_ref, *, act):
    acc = p_ref[0] + p_ref[1]
    h = jnp.dot(acc, wt_ref[...], preferred_element_type=jnp.float32)
    h = h + b_ref[...]
    if act:
        h = jnp.maximum(h, 0.0)
    o_ref[...] = h


def _tc_conv(p, wt, b2d, act):
    blk = 1024
    return pl.pallas_call(
        functools.partial(_tc_conv_body, act=act),
        grid=(NP // blk,),
        in_specs=[
            pl.BlockSpec((NC, blk, D), lambda i: (0, i, 0)),
            pl.BlockSpec((D, D), lambda i: (0, 0)),
            pl.BlockSpec((1, D), lambda i: (0, 0)),
        ],
        out_specs=pl.BlockSpec((blk, D), lambda i: (i, 0)),
        out_shape=jax.ShapeDtypeStruct((NP, D), jnp.float32),
    )(p, wt, b2d)


def _tc_final_body(p_ref, w3t_ref, b3_ref, wp1t_ref, bp1_ref, wp2_ref,
                   bp2_ref, scores_ref, h_ref):
    acc = p_ref[0] + p_ref[1]
    h = jnp.dot(acc, w3t_ref[...], preferred_element_type=jnp.float32)
    h = h + b3_ref[...]
    h_ref[...] = h
    pooled = jnp.sum(h[:N], axis=0, keepdims=True) / N
    a = jnp.dot(pooled, wp1t_ref[...],
                preferred_element_type=jnp.float32) + bp1_ref[...]
    a = jnp.maximum(a, 0.0)
    s = jnp.sum(a * wp2_ref[...]) + bp2_ref[0, 0]
    s = 1.0 / (1.0 + jnp.exp(-s))
    scores_ref[...] = jnp.full((NP, 1), s, jnp.float32)


def _tc_final(p, w3t, b3_2d, wp1t, bp1_2d, wp2, bp2_2d):
    return pl.pallas_call(
        _tc_final_body,
        out_shape=(
            jax.ShapeDtypeStruct((NP, 1), jnp.float32),
            jax.ShapeDtypeStruct((NP, D), jnp.float32),
        ),
    )(p, w3t, b3_2d, wp1t, bp1_2d, wp2, bp2_2d)


def kernel(x, edge_index, edge_weight, W1, b1, W2, b2, W3, b3,
           Wp1, bp1, Wp2, bp2):
    row3 = edge_index[0].astype(jnp.int32).reshape(NW, NCHUNK, CH)
    col3 = edge_index[1].astype(jnp.int32).reshape(NW, NCHUNK, CH)
    w3e = edge_weight.reshape(NW, NCHUNK, CH)
    zeros = jnp.zeros((NP, D), jnp.float32)

    w1t = W1.T
    w2t = W2.T
    w3t = W3.T
    wp1t = Wp1.T

    p = _sc_aggregate(x, row3, col3, w3e, zeros)
    h = _tc_conv(p, w1t, b1.reshape(1, D), act=True)
    p = _sc_aggregate(h, row3, col3, w3e, zeros)
    h = _tc_conv(p, w2t, b2.reshape(1, D), act=True)
    p = _sc_aggregate(h, row3, col3, w3e, zeros)
    scores, hout = _tc_final(p, w3t, b3.reshape(1, D), wp1t,
                             bp1.reshape(1, D // 2), Wp2,
                             bp2.reshape(1, 1))
    return (scores[:N], hout[:N])
